# Initial kernel scaffold; baseline (speedup 1.0000x reference)
import jax
import jax.numpy as jnp
from jax.experimental import pallas as pl


def kernel(q, S, qAV_num, SAV_num, shot_num):
    raise NotImplementedError("write your pallas kernel here")


# speedup vs baseline: 15.8651x; 15.8651x over previous
import jax
import jax.numpy as jnp
from jax.experimental import pallas as pl
from jax.experimental.pallas import tpu as pltpu

_HW = 196
_C = 384
_NCLS = 10
_MPAD = 1024
_NEIGHBOR_K = 3


def _knn_body(q_ref, s_ref, o_ref):
    qb = q_ref[0]
    rq = jax.lax.rsqrt(jnp.sum(qb * qb, axis=1, keepdims=True))
    sc = s_ref[0]
    ss = jnp.sum(sc * sc, axis=1)
    rs = jnp.where(ss > 0, jax.lax.rsqrt(ss), 0.0)
    a = jax.lax.dot_general(
        qb, sc, (((1,), (1,)), ((), ())), preferred_element_type=jnp.float32
    )
    a = a * rq * rs[None, :]
    iota = jax.lax.broadcasted_iota(jnp.int32, a.shape, 0)
    m1 = jnp.max(a, axis=0)
    i1 = jnp.argmax(a, axis=0)
    a = jnp.where(iota == i1[None, :], -jnp.inf, a)
    m2 = jnp.max(a, axis=0)
    i2 = jnp.argmax(a, axis=0)
    a = jnp.where(iota == i2[None, :], -jnp.inf, a)
    m3 = jnp.max(a, axis=0)
    c = pl.program_id(0)
    b = pl.program_id(1)
    o_ref[c, b] = jnp.sum(m1 + m2 + m3)


def kernel(q, S, qAV_num, SAV_num, shot_num):
    B = q.shape[0]
    q2 = q.reshape(B, _C, _HW).transpose(0, 2, 1)
    s2 = S.reshape(S.shape[0], _C, _HW).transpose(0, 2, 1)
    s2 = s2.reshape(_NCLS, -1, _C)
    s2 = jnp.pad(s2, ((0, 0), (0, _MPAD - s2.shape[1]), (0, 0)))

    out = pl.pallas_call(
        _knn_body,
        grid=(_NCLS, B),
        in_specs=[
            pl.BlockSpec((1, _HW, _C), lambda c, b: (b, 0, 0)),
            pl.BlockSpec((1, _MPAD, _C), lambda c, b: (c, 0, 0)),
        ],
        out_specs=pl.BlockSpec(memory_space=pltpu.SMEM),
        out_shape=jax.ShapeDtypeStruct((_NCLS, B), jnp.float32),
    )(q2, s2)
    return out.T
